# Initial kernel scaffold; baseline (speedup 1.0000x reference)
import jax
import jax.numpy as jnp
from jax.experimental import pallas as pl


def kernel(x, edge_index, W1, b1, W2, b2):
    raise NotImplementedError("write your pallas kernel here")


# speedup vs baseline: 118.4923x; 118.4923x over previous
import functools

import jax
import jax.numpy as jnp
from jax import lax
from jax.experimental import pallas as pl
from jax.experimental.pallas import tpu as pltpu
from jax.experimental.pallas import tpu_sc as plsc

ROW = 128


def _row_ranges(n_rows, nw):
    base = n_rows // nw
    extra = n_rows - base * nw
    return base, extra


def _make_deg_kernel(n_rows, n_pad, nc, ns):
    nw = nc * ns
    base, extra = _row_ranges(n_rows, nw)
    stripe = n_pad // ns
    mesh = plsc.VectorSubcoreMesh(core_axis_name="c", subcore_axis_name="s")

    @functools.partial(
        pl.kernel,
        mesh=mesh,
        out_type=jax.ShapeDtypeStruct((nc * n_pad,), jnp.float32),
        scratch_types=[
            pltpu.VMEM((ROW,), jnp.int32),
            pltpu.VMEM((ROW,), jnp.int32),
            pltpu.VMEM((ROW,), jnp.float32),
            pltpu.VMEM((stripe,), jnp.float32),
            pltpu.VMEM_SHARED((n_pad,), jnp.float32),
            pltpu.SemaphoreType.DMA,
            pltpu.SemaphoreType.DMA,
        ],
    )
    def deg_kernel(dst_hbm, out_hbm, d0, d1, ones_v, zbuf, acc, sm0, sm1):
        c = lax.axis_index("c")
        s = lax.axis_index("s")
        wid = s * nc + c
        r0 = wid * base

        def fill_ones(i, carry):
            ones_v[pl.ds(i * 16, 16)] = jnp.full((16,), 1.0, jnp.float32)
            return carry

        lax.fori_loop(0, ROW // 16, fill_ones, 0)

        def fill_zero(i, carry):
            zbuf[pl.ds(i * 16, 16)] = jnp.zeros((16,), jnp.float32)
            return carry

        lax.fori_loop(0, stripe // 16, fill_zero, 0)
        pltpu.sync_copy(zbuf, acc.at[pl.ds(s * stripe, stripe)])
        plsc.subcore_barrier()

        last = r0 + base - 1
        pltpu.async_copy(dst_hbm.at[r0], d0, sm0)
        pltpu.async_copy(dst_hbm.at[r0 + 1], d1, sm1)

        def body(jj, carry):
            j0 = r0 + jj * 2
            pltpu.make_async_copy(dst_hbm.at[j0], d0, sm0).wait()
            pltpu.sync_copy(ones_v, acc.at[d0], add=True)
            pltpu.async_copy(dst_hbm.at[jnp.minimum(j0 + 2, last)], d0, sm0)
            pltpu.make_async_copy(dst_hbm.at[j0 + 1], d1, sm1).wait()
            pltpu.sync_copy(ones_v, acc.at[d1], add=True)
            pltpu.async_copy(dst_hbm.at[jnp.minimum(j0 + 3, last)], d1, sm1)
            return carry

        lax.fori_loop(0, base // 2, body, 0)
        pltpu.make_async_copy(dst_hbm.at[last], d0, sm0).wait()
        pltpu.sync_copy(ones_v, acc.at[d0], add=True)
        pltpu.make_async_copy(dst_hbm.at[last], d1, sm1).wait()

        @pl.when(wid < extra)
        def _():
            rx = nw * base + wid
            pltpu.async_copy(dst_hbm.at[rx], d1, sm1)
            pltpu.make_async_copy(dst_hbm.at[rx], d1, sm1).wait()
            pltpu.sync_copy(ones_v, acc.at[d1], add=True)

        plsc.subcore_barrier()
        pltpu.sync_copy(acc.at[pl.ds(s * stripe, stripe)], zbuf)
        pltpu.sync_copy(zbuf, out_hbm.at[pl.ds(c * n_pad + s * stripe, stripe)])

    return deg_kernel


def _make_agg_kernel(n_rows, n_pad, nc, ns):
    nw = nc * ns
    base, extra = _row_ranges(n_rows, nw)
    stripe = n_pad // ns
    mesh = plsc.VectorSubcoreMesh(core_axis_name="c", subcore_axis_name="s")

    @functools.partial(
        pl.kernel,
        mesh=mesh,
        out_type=jax.ShapeDtypeStruct((nc * n_pad,), jnp.float32),
        compiler_params=pltpu.CompilerParams(needs_layout_passes=False),
        scratch_types=[
            pltpu.VMEM((n_pad,), jnp.float32),
            pltpu.VMEM((ROW,), jnp.int32),
            pltpu.VMEM((ROW,), jnp.int32),
            pltpu.VMEM((ROW,), jnp.int32),
            pltpu.VMEM((ROW,), jnp.int32),
            pltpu.VMEM((ROW,), jnp.float32),
            pltpu.VMEM((ROW,), jnp.float32),
            pltpu.VMEM((stripe,), jnp.float32),
            pltpu.VMEM_SHARED((n_pad,), jnp.float32),
            pltpu.SemaphoreType.DMA,
            pltpu.SemaphoreType.DMA,
        ],
    )
    def agg_kernel(src_hbm, dst_hbm, u_hbm, out_hbm,
                   utab, s0b, s1b, d0, d1, v0, v1, zbuf, acc, sm0, sm1):
        c = lax.axis_index("c")
        s = lax.axis_index("s")
        wid = s * nc + c
        r0 = wid * base

        pltpu.sync_copy(u_hbm, utab)

        def fill_zero(i, carry):
            zbuf[pl.ds(i * 16, 16)] = jnp.zeros((16,), jnp.float32)
            return carry

        lax.fori_loop(0, stripe // 16, fill_zero, 0)
        pltpu.sync_copy(zbuf, acc.at[pl.ds(s * stripe, stripe)])
        plsc.subcore_barrier()

        def process(srcb, dstb, valsb):
            for g in range(ROW // 16):
                idx = srcb[pl.ds(g * 16, 16)]
                valsb[pl.ds(g * 16, 16)] = plsc.load_gather(utab, [idx])
            pltpu.sync_copy(valsb, acc.at[dstb], add=True)

        last = r0 + base - 1
        pltpu.async_copy(src_hbm.at[r0], s0b, sm0)
        pltpu.async_copy(dst_hbm.at[r0], d0, sm0)
        pltpu.async_copy(src_hbm.at[r0 + 1], s1b, sm1)
        pltpu.async_copy(dst_hbm.at[r0 + 1], d1, sm1)

        def body(jj, carry):
            j0 = r0 + jj * 2
            pltpu.make_async_copy(src_hbm.at[j0], s0b, sm0).wait()
            pltpu.make_async_copy(dst_hbm.at[j0], d0, sm0).wait()
            process(s0b, d0, v0)
            nxt = jnp.minimum(j0 + 2, last)
            pltpu.async_copy(src_hbm.at[nxt], s0b, sm0)
            pltpu.async_copy(dst_hbm.at[nxt], d0, sm0)
            pltpu.make_async_copy(src_hbm.at[j0 + 1], s1b, sm1).wait()
            pltpu.make_async_copy(dst_hbm.at[j0 + 1], d1, sm1).wait()
            process(s1b, d1, v1)
            nxt1 = jnp.minimum(j0 + 3, last)
            pltpu.async_copy(src_hbm.at[nxt1], s1b, sm1)
            pltpu.async_copy(dst_hbm.at[nxt1], d1, sm1)
            return carry

        lax.fori_loop(0, base // 2, body, 0)
        pltpu.make_async_copy(src_hbm.at[last], s0b, sm0).wait()
        pltpu.make_async_copy(dst_hbm.at[last], d0, sm0).wait()
        process(s0b, d0, v0)
        pltpu.make_async_copy(src_hbm.at[last], s1b, sm1).wait()
        pltpu.make_async_copy(dst_hbm.at[last], d1, sm1).wait()

        @pl.when(wid < extra)
        def _():
            rx = nw * base + wid
            pltpu.async_copy(src_hbm.at[rx], s1b, sm1)
            pltpu.async_copy(dst_hbm.at[rx], d1, sm1)
            pltpu.make_async_copy(src_hbm.at[rx], s1b, sm1).wait()
            pltpu.make_async_copy(dst_hbm.at[rx], d1, sm1).wait()
            process(s1b, d1, v1)

        plsc.subcore_barrier()
        pltpu.sync_copy(acc.at[pl.ds(s * stripe, stripe)], zbuf)
        pltpu.sync_copy(zbuf, out_hbm.at[pl.ds(c * n_pad + s * stripe, stripe)])

    return agg_kernel


def _stage1_body(p_ref, x_ref, dinv_ref, u_ref):
    deg = p_ref[0] + p_ref[1] + 1.0
    dinv = lax.rsqrt(deg)
    dinv_ref[...] = dinv
    u_ref[...] = x_ref[...] * dinv


def _stage2_body(p_ref, x_ref, dinv_ref, w1_ref, b1_ref, w2_ref, t_ref, v_ref):
    dinv = dinv_ref[...]
    s1 = dinv * (p_ref[0] + p_ref[1]) + x_ref[...] * dinv * dinv
    t = jnp.zeros_like(s1)
    for k in range(16):
        t = t + jnp.maximum(s1 * w1_ref[0, k] + b1_ref[0, k], 0.0) * w2_ref[0, k]
    t_ref[...] = t
    v_ref[...] = t * dinv


def _stage3_body(p_ref, t_ref, dinv_ref, b2_ref, o_ref):
    dinv = dinv_ref[...]
    t = t_ref[...]
    o_ref[...] = dinv * (p_ref[0] + p_ref[1]) + t * dinv * dinv + b2_ref[0]


def kernel(x, edge_index, W1, b1, W2, b2):
    n = x.shape[0]
    e = edge_index.shape[1]
    assert e % ROW == 0
    n_rows = e // ROW

    info = plsc.get_sparse_core_info()
    nc, ns = info.num_cores, info.num_subcores

    stripe = -(-n // (ns * 8)) * 8
    n_pad = ns * stripe
    rn = n_pad // 128

    ei = edge_index.astype(jnp.int32)
    src_r = ei[0].reshape(n_rows, ROW)
    dst_r = ei[1].reshape(n_rows, ROW)
    x2d = jnp.pad(x[:, 0], (0, n_pad - n)).reshape(rn, 128)

    deg_k = _make_deg_kernel(n_rows, n_pad, nc, ns)
    agg_k = _make_agg_kernel(n_rows, n_pad, nc, ns)

    degp = deg_k(dst_r).reshape(nc, rn, 128)

    smem = pl.BlockSpec(memory_space=pltpu.SMEM)
    f32 = jnp.float32
    dinv2d, u2d = pl.pallas_call(
        _stage1_body,
        out_shape=[jax.ShapeDtypeStruct((rn, 128), f32)] * 2,
    )(degp, x2d)

    s1p = agg_k(src_r, dst_r, u2d.reshape(n_pad)).reshape(nc, rn, 128)

    t2d, v2d = pl.pallas_call(
        _stage2_body,
        in_specs=[pl.BlockSpec(), pl.BlockSpec(), pl.BlockSpec(),
                  smem, smem, smem],
        out_shape=[jax.ShapeDtypeStruct((rn, 128), f32)] * 2,
    )(s1p, x2d, dinv2d, W1.reshape(1, 16), b1.reshape(1, 16), W2.reshape(1, 16))

    s2p = agg_k(src_r, dst_r, v2d.reshape(n_pad)).reshape(nc, rn, 128)

    o2d = pl.pallas_call(
        _stage3_body,
        in_specs=[pl.BlockSpec(), pl.BlockSpec(), pl.BlockSpec(), smem],
        out_shape=jax.ShapeDtypeStruct((rn, 128), f32),
    )(s2p, t2d, dinv2d, b2)

    return o2d.reshape(n_pad)[:n]


# speedup vs baseline: 444.7571x; 3.7535x over previous
import functools

import jax
import jax.numpy as jnp
from jax import lax
from jax.experimental import pallas as pl
from jax.experimental.pallas import tpu as pltpu
from jax.experimental.pallas import tpu_sc as plsc

ROW = 128
BLK = 8
DEPTH = 8
LEAD = 4
NSLOT = 2 * BLK


def _tile_blocks(n_blocks, nw):
    base = n_blocks // nw
    extra = n_blocks - base * nw
    return base, extra


def _block_loop(base, extra, wid, issue_dma, wait_dma, process, wait_scat):
    def do_block(k, p, parity, skip_slot_wait=False, guard_dma=False):
        def issue():
            issue_dma(k + LEAD, (p + LEAD) % DEPTH)

        if guard_dma:
            nblk_w = base + jnp.where(wid < extra, 1, 0)
            pl.when(k + LEAD < nblk_w)(issue)
        else:
            issue()
        wait_dma(p)
        process(p, parity, skip_slot_wait)

    for k in range(LEAD):
        issue_dma(jnp.int32(k), k)
    yield

    for k in range(LEAD):
        do_block(jnp.int32(k), k, k % 2, skip_slot_wait=k < 2)

    def body(t, carry):
        for pp in range(DEPTH):
            do_block(t * DEPTH + LEAD + pp, (LEAD + pp) % DEPTH, pp % 2)
        return carry

    n_main = (base - LEAD) // DEPTH
    lax.fori_loop(0, n_main, body, 0)

    for q in range(LEAD + n_main * DEPTH, base + 1):
        blk_fn = functools.partial(do_block, jnp.int32(q), q % DEPTH, q % 2,
                                   False, True)
        if q == base:
            pl.when(wid < extra)(blk_fn)
        else:
            blk_fn()

    for sv in range(NSLOT):
        wait_scat(sv)
    yield


def _make_deg_kernel(n_blocks, n_pad, nc, ns):
    nw = nc * ns
    base, extra = _tile_blocks(n_blocks, nw)
    stripe = n_pad // ns
    mesh = plsc.VectorSubcoreMesh(core_axis_name="c", subcore_axis_name="s")

    @functools.partial(
        pl.kernel,
        mesh=mesh,
        out_type=jax.ShapeDtypeStruct((nc * n_pad,), jnp.float32),
        scratch_types=(
            [pltpu.VMEM((BLK, ROW), jnp.int32) for _ in range(DEPTH)]
            + [pltpu.VMEM((ROW,), jnp.float32),
               pltpu.VMEM((stripe,), jnp.float32),
               pltpu.VMEM_SHARED((n_pad,), jnp.float32)]
            + [pltpu.SemaphoreType.DMA for _ in range(DEPTH + NSLOT)]
        ),
    )
    def deg_kernel(dst_hbm, out_hbm, *scratch):
        dbuf = scratch[:DEPTH]
        ones, zbuf, acc = scratch[DEPTH:DEPTH + 3]
        edma = scratch[DEPTH + 3:DEPTH + 3 + DEPTH]
        ssem = scratch[DEPTH + 3 + DEPTH:]

        c = lax.axis_index("c")
        s = lax.axis_index("s")
        wid = s * nc + c

        for g in range(ROW // 16):
            ones[pl.ds(g * 16, 16)] = jnp.full((16,), 1.0, jnp.float32)

        def fill_zero(i, carry):
            zbuf[pl.ds(i * 16, 16)] = jnp.zeros((16,), jnp.float32)
            return carry

        lax.fori_loop(0, stripe // 16, fill_zero, 0)
        pltpu.sync_copy(zbuf, acc.at[pl.ds(s * stripe, stripe)])

        def issue_dma(k, p):
            blk = wid + k * nw
            pltpu.async_copy(dst_hbm.at[pl.ds(blk * BLK, BLK)], dbuf[p],
                             edma[p])

        def wait_dma(p):
            pltpu.make_async_copy(dst_hbm.at[pl.ds(0, BLK)], dbuf[p],
                                  edma[p]).wait()

        def wait_scat(sv):
            pltpu.make_async_copy(ones, acc.at[dbuf[0].at[0]],
                                  ssem[sv]).wait()

        def process(p, parity, skip_slot_wait):
            for r in range(BLK):
                sv = parity * BLK + r
                if not skip_slot_wait:
                    wait_scat(sv)
                pltpu.async_copy(ones, acc.at[dbuf[p].at[r]], ssem[sv],
                                 add=True)

        loop = _block_loop(base, extra, wid, issue_dma, wait_dma, process,
                           wait_scat)
        next(loop)
        plsc.subcore_barrier()
        next(loop, None)

        plsc.subcore_barrier()
        pltpu.sync_copy(acc.at[pl.ds(s * stripe, stripe)], zbuf)
        pltpu.sync_copy(zbuf, out_hbm.at[pl.ds(c * n_pad + s * stripe, stripe)])

    return deg_kernel


def _make_agg_kernel(n_blocks, n_pad, nc, ns):
    nw = nc * ns
    base, extra = _tile_blocks(n_blocks, nw)
    stripe = n_pad // ns
    mesh = plsc.VectorSubcoreMesh(core_axis_name="c", subcore_axis_name="s")

    @functools.partial(
        pl.kernel,
        mesh=mesh,
        out_type=jax.ShapeDtypeStruct((nc * n_pad,), jnp.float32),
        compiler_params=pltpu.CompilerParams(needs_layout_passes=False),
        scratch_types=(
            [pltpu.VMEM((n_pad,), jnp.float32)]
            + [pltpu.VMEM((2, BLK, ROW), jnp.int32) for _ in range(DEPTH)]
            + [pltpu.VMEM((ROW,), jnp.float32) for _ in range(NSLOT)]
            + [pltpu.VMEM((stripe,), jnp.float32),
               pltpu.VMEM_SHARED((n_pad,), jnp.float32)]
            + [pltpu.SemaphoreType.DMA for _ in range(1 + DEPTH + NSLOT)]
        ),
    )
    def agg_kernel(e_hbm, u_hbm, out_hbm, *scratch):
        utab = scratch[0]
        ebuf = scratch[1:1 + DEPTH]
        vbuf = scratch[1 + DEPTH:1 + DEPTH + NSLOT]
        zbuf, acc = scratch[1 + DEPTH + NSLOT:3 + DEPTH + NSLOT]
        usem = scratch[3 + DEPTH + NSLOT]
        edma = scratch[4 + DEPTH + NSLOT:4 + 2 * DEPTH + NSLOT]
        ssem = scratch[4 + 2 * DEPTH + NSLOT:]

        c = lax.axis_index("c")
        s = lax.axis_index("s")
        wid = s * nc + c

        pltpu.async_copy(u_hbm, utab, usem)

        def fill_zero(i, carry):
            zbuf[pl.ds(i * 16, 16)] = jnp.zeros((16,), jnp.float32)
            return carry

        lax.fori_loop(0, stripe // 16, fill_zero, 0)
        pltpu.sync_copy(zbuf, acc.at[pl.ds(s * stripe, stripe)])

        def issue_dma(k, p):
            blk = wid + k * nw
            pltpu.async_copy(e_hbm.at[:, pl.ds(blk * BLK, BLK)], ebuf[p],
                             edma[p])

        def wait_dma(p):
            pltpu.make_async_copy(e_hbm.at[:, pl.ds(0, BLK)], ebuf[p],
                                  edma[p]).wait()

        def wait_scat(sv):
            pltpu.make_async_copy(vbuf[sv], acc.at[ebuf[0].at[1, 0]],
                                  ssem[sv]).wait()

        def process(p, parity, skip_slot_wait):
            for r in range(BLK):
                sv = parity * BLK + r
                if not skip_slot_wait:
                    wait_scat(sv)
                for g in range(ROW // 16):
                    idx = ebuf[p][0, r, pl.ds(g * 16, 16)]
                    vbuf[sv][pl.ds(g * 16, 16)] = plsc.load_gather(
                        utab, [idx])
                pltpu.async_copy(vbuf[sv], acc.at[ebuf[p].at[1, r]], ssem[sv],
                                 add=True)

        loop = _block_loop(base, extra, wid, issue_dma, wait_dma, process,
                           wait_scat)
        next(loop)
        pltpu.make_async_copy(u_hbm, utab, usem).wait()
        plsc.subcore_barrier()
        next(loop, None)

        plsc.subcore_barrier()
        pltpu.sync_copy(acc.at[pl.ds(s * stripe, stripe)], zbuf)
        pltpu.sync_copy(zbuf, out_hbm.at[pl.ds(c * n_pad + s * stripe, stripe)])

    return agg_kernel


def _stage1_body(p_ref, x_ref, dinv_ref, u_ref):
    deg = p_ref[0] + p_ref[1] + 1.0
    dinv = lax.rsqrt(deg)
    dinv_ref[...] = dinv
    u_ref[...] = x_ref[...] * dinv


def _stage2_body(p_ref, x_ref, dinv_ref, w1_ref, b1_ref, w2_ref, t_ref, v_ref):
    dinv = dinv_ref[...]
    s1 = dinv * (p_ref[0] + p_ref[1]) + x_ref[...] * dinv * dinv
    t = jnp.zeros_like(s1)
    for k in range(16):
        t = t + jnp.maximum(s1 * w1_ref[0, k] + b1_ref[0, k], 0.0) * w2_ref[0, k]
    t_ref[...] = t
    v_ref[...] = t * dinv


def _stage3_body(p_ref, t_ref, dinv_ref, b2_ref, o_ref):
    dinv = dinv_ref[...]
    t = t_ref[...]
    o_ref[...] = dinv * (p_ref[0] + p_ref[1]) + t * dinv * dinv + b2_ref[0]


def kernel(x, edge_index, W1, b1, W2, b2):
    n = x.shape[0]
    e = edge_index.shape[1]
    assert e % (ROW * BLK) == 0
    n_rows = e // ROW
    n_blocks = n_rows // BLK

    info = plsc.get_sparse_core_info()
    nc, ns = info.num_cores, info.num_subcores

    stripe = -(-n // (ns * 8)) * 8
    n_pad = ns * stripe
    rn = n_pad // 128

    ei = edge_index.astype(jnp.int32)
    e3 = ei.reshape(2, n_rows, ROW)
    x2d = jnp.pad(x[:, 0], (0, n_pad - n)).reshape(rn, 128)

    deg_k = _make_deg_kernel(n_blocks, n_pad, nc, ns)
    agg_k = _make_agg_kernel(n_blocks, n_pad, nc, ns)

    degp = deg_k(e3[1]).reshape(nc, rn, 128)

    smem = pl.BlockSpec(memory_space=pltpu.SMEM)
    f32 = jnp.float32
    dinv2d, u2d = pl.pallas_call(
        _stage1_body,
        out_shape=[jax.ShapeDtypeStruct((rn, 128), f32)] * 2,
    )(degp, x2d)

    s1p = agg_k(e3, u2d.reshape(n_pad)).reshape(nc, rn, 128)

    t2d, v2d = pl.pallas_call(
        _stage2_body,
        in_specs=[pl.BlockSpec(), pl.BlockSpec(), pl.BlockSpec(),
                  smem, smem, smem],
        out_shape=[jax.ShapeDtypeStruct((rn, 128), f32)] * 2,
    )(s1p, x2d, dinv2d, W1.reshape(1, 16), b1.reshape(1, 16), W2.reshape(1, 16))

    s2p = agg_k(e3, v2d.reshape(n_pad)).reshape(nc, rn, 128)

    o2d = pl.pallas_call(
        _stage3_body,
        in_specs=[pl.BlockSpec(), pl.BlockSpec(), pl.BlockSpec(), smem],
        out_shape=jax.ShapeDtypeStruct((rn, 128), f32),
    )(s2p, t2d, dinv2d, b2)

    return o2d.reshape(n_pad)[:n]


# speedup vs baseline: 464.8386x; 1.0452x over previous
import functools

import jax
import jax.numpy as jnp
from jax import lax
from jax.experimental import pallas as pl
from jax.experimental.pallas import tpu as pltpu
from jax.experimental.pallas import tpu_sc as plsc

ROW = 128
BLK = 8
DEPTH = 8
LEAD = 4
NSLOT = 2 * BLK


def _tile_blocks(n_blocks, nw):
    base = n_blocks // nw
    extra = n_blocks - base * nw
    return base, extra


def _block_loop(base, extra, wid, issue_dma, wait_dma, process, wait_scat):
    def do_block(k, p, parity, skip_slot_wait=False, guard_dma=False):
        def issue():
            issue_dma(k + LEAD, (p + LEAD) % DEPTH)

        if guard_dma:
            nblk_w = base + jnp.where(wid < extra, 1, 0)
            pl.when(k + LEAD < nblk_w)(issue)
        else:
            issue()
        wait_dma(p)
        process(p, parity, skip_slot_wait)

    for k in range(LEAD):
        issue_dma(jnp.int32(k), k)
    yield

    for k in range(LEAD):
        do_block(jnp.int32(k), k, k % 2, skip_slot_wait=k < 2)

    def body(t, carry):
        for pp in range(DEPTH):
            do_block(t * DEPTH + LEAD + pp, (LEAD + pp) % DEPTH, pp % 2)
        return carry

    n_main = (base - LEAD) // DEPTH
    lax.fori_loop(0, n_main, body, 0)

    for q in range(LEAD + n_main * DEPTH, base + 1):
        blk_fn = functools.partial(do_block, jnp.int32(q), q % DEPTH, q % 2,
                                   False, True)
        if q == base:
            pl.when(wid < extra)(blk_fn)
        else:
            blk_fn()

    for sv in range(NSLOT):
        wait_scat(sv)
    yield


def _make_deg_kernel(n_blocks, n_pad, nc, ns):
    nw = nc * ns
    base, extra = _tile_blocks(n_blocks, nw)
    stripe = n_pad // ns
    mesh = plsc.VectorSubcoreMesh(core_axis_name="c", subcore_axis_name="s")

    @functools.partial(
        pl.kernel,
        mesh=mesh,
        out_type=jax.ShapeDtypeStruct((nc * n_pad,), jnp.float32),
        scratch_types=(
            [pltpu.VMEM((BLK, ROW), jnp.int32) for _ in range(DEPTH)]
            + [pltpu.VMEM((ROW,), jnp.float32),
               pltpu.VMEM((stripe,), jnp.float32),
               pltpu.VMEM_SHARED((n_pad,), jnp.float32)]
            + [pltpu.SemaphoreType.DMA for _ in range(DEPTH + NSLOT)]
        ),
    )
    def deg_kernel(e_hbm, out_hbm, *scratch):
        dbuf = scratch[:DEPTH]
        ones, zbuf, acc = scratch[DEPTH:DEPTH + 3]
        edma = scratch[DEPTH + 3:DEPTH + 3 + DEPTH]
        ssem = scratch[DEPTH + 3 + DEPTH:]

        c = lax.axis_index("c")
        s = lax.axis_index("s")
        wid = s * nc + c

        for g in range(ROW // 16):
            ones[pl.ds(g * 16, 16)] = jnp.full((16,), 1.0, jnp.float32)

        def fill_zero(i, carry):
            zbuf[pl.ds(i * 16, 16)] = jnp.zeros((16,), jnp.float32)
            return carry

        lax.fori_loop(0, stripe // 16, fill_zero, 0)
        pltpu.sync_copy(zbuf, acc.at[pl.ds(s * stripe, stripe)])

        def issue_dma(k, p):
            blk = wid + k * nw
            pltpu.async_copy(e_hbm.at[1, pl.ds(blk * BLK, BLK)], dbuf[p],
                             edma[p])

        def wait_dma(p):
            pltpu.make_async_copy(e_hbm.at[1, pl.ds(0, BLK)], dbuf[p],
                                  edma[p]).wait()

        def wait_scat(sv):
            pltpu.make_async_copy(ones, acc.at[dbuf[0].at[0]],
                                  ssem[sv]).wait()

        def process(p, parity, skip_slot_wait):
            for r in range(BLK):
                sv = parity * BLK + r
                if not skip_slot_wait:
                    wait_scat(sv)
                pltpu.async_copy(ones, acc.at[dbuf[p].at[r]], ssem[sv],
                                 add=True)

        loop = _block_loop(base, extra, wid, issue_dma, wait_dma, process,
                           wait_scat)
        next(loop)
        plsc.subcore_barrier()
        next(loop, None)

        plsc.subcore_barrier()
        pltpu.sync_copy(acc.at[pl.ds(s * stripe, stripe)], zbuf)
        pltpu.sync_copy(zbuf, out_hbm.at[pl.ds(c * n_pad + s * stripe, stripe)])

    return deg_kernel


def _make_agg_kernel(n_blocks, n_pad, nc, ns):
    nw = nc * ns
    base, extra = _tile_blocks(n_blocks, nw)
    stripe = n_pad // ns
    mesh = plsc.VectorSubcoreMesh(core_axis_name="c", subcore_axis_name="s")

    @functools.partial(
        pl.kernel,
        mesh=mesh,
        out_type=jax.ShapeDtypeStruct((nc * n_pad,), jnp.float32),
        compiler_params=pltpu.CompilerParams(needs_layout_passes=False),
        scratch_types=(
            [pltpu.VMEM((n_pad,), jnp.float32)]
            + [pltpu.VMEM((2, BLK, ROW), jnp.int32) for _ in range(DEPTH)]
            + [pltpu.VMEM((ROW,), jnp.float32) for _ in range(NSLOT)]
            + [pltpu.VMEM((stripe,), jnp.float32),
               pltpu.VMEM_SHARED((n_pad,), jnp.float32)]
            + [pltpu.SemaphoreType.DMA for _ in range(1 + DEPTH + NSLOT)]
        ),
    )
    def agg_kernel(e_hbm, u_hbm, out_hbm, *scratch):
        utab = scratch[0]
        ebuf = scratch[1:1 + DEPTH]
        vbuf = scratch[1 + DEPTH:1 + DEPTH + NSLOT]
        zbuf, acc = scratch[1 + DEPTH + NSLOT:3 + DEPTH + NSLOT]
        usem = scratch[3 + DEPTH + NSLOT]
        edma = scratch[4 + DEPTH + NSLOT:4 + 2 * DEPTH + NSLOT]
        ssem = scratch[4 + 2 * DEPTH + NSLOT:]

        c = lax.axis_index("c")
        s = lax.axis_index("s")
        wid = s * nc + c

        pltpu.async_copy(u_hbm, utab, usem)

        def fill_zero(i, carry):
            zbuf[pl.ds(i * 16, 16)] = jnp.zeros((16,), jnp.float32)
            return carry

        lax.fori_loop(0, stripe // 16, fill_zero, 0)
        pltpu.sync_copy(zbuf, acc.at[pl.ds(s * stripe, stripe)])

        def issue_dma(k, p):
            blk = wid + k * nw
            pltpu.async_copy(e_hbm.at[:, pl.ds(blk * BLK, BLK)], ebuf[p],
                             edma[p])

        def wait_dma(p):
            pltpu.make_async_copy(e_hbm.at[:, pl.ds(0, BLK)], ebuf[p],
                                  edma[p]).wait()

        def wait_scat(sv):
            pltpu.make_async_copy(vbuf[sv], acc.at[ebuf[0].at[1, 0]],
                                  ssem[sv]).wait()

        def process(p, parity, skip_slot_wait):
            for r in range(BLK):
                sv = parity * BLK + r
                if not skip_slot_wait:
                    wait_scat(sv)
                for g in range(ROW // 16):
                    idx = ebuf[p][0, r, pl.ds(g * 16, 16)]
                    vbuf[sv][pl.ds(g * 16, 16)] = plsc.load_gather(
                        utab, [idx])
                pltpu.async_copy(vbuf[sv], acc.at[ebuf[p].at[1, r]], ssem[sv],
                                 add=True)

        loop = _block_loop(base, extra, wid, issue_dma, wait_dma, process,
                           wait_scat)
        next(loop)
        pltpu.make_async_copy(u_hbm, utab, usem).wait()
        plsc.subcore_barrier()
        next(loop, None)

        plsc.subcore_barrier()
        pltpu.sync_copy(acc.at[pl.ds(s * stripe, stripe)], zbuf)
        pltpu.sync_copy(zbuf, out_hbm.at[pl.ds(c * n_pad + s * stripe, stripe)])

    return agg_kernel


def _stage1_body(p_ref, x_ref, dinv_ref, u_ref):
    deg = p_ref[0] + p_ref[1] + 1.0
    dinv = lax.rsqrt(deg)
    dinv_ref[...] = dinv
    u_ref[...] = x_ref[...] * dinv


def _stage2_body(p_ref, x_ref, dinv_ref, w1_ref, b1_ref, w2_ref, t_ref, v_ref):
    dinv = dinv_ref[...]
    s1 = dinv * (p_ref[0] + p_ref[1]) + x_ref[...] * dinv * dinv
    t = jnp.zeros_like(s1)
    for k in range(16):
        t = t + jnp.maximum(s1 * w1_ref[0, k] + b1_ref[0, k], 0.0) * w2_ref[0, k]
    t_ref[...] = t
    v_ref[...] = t * dinv


def _stage3_body(p_ref, t_ref, dinv_ref, b2_ref, o_ref):
    dinv = dinv_ref[...]
    t = t_ref[...]
    o_ref[...] = dinv * (p_ref[0] + p_ref[1]) + t * dinv * dinv + b2_ref[0]


def kernel(x, edge_index, W1, b1, W2, b2):
    n = x.shape[0]
    e = edge_index.shape[1]
    assert e % (ROW * BLK) == 0
    n_rows = e // ROW
    n_blocks = n_rows // BLK

    info = plsc.get_sparse_core_info()
    nc, ns = info.num_cores, info.num_subcores

    stripe = -(-n // (ns * 8)) * 8
    n_pad = ns * stripe
    rn = n_pad // 128

    ei = edge_index.astype(jnp.int32)
    e3 = ei.reshape(2, n_rows, ROW)
    x2d = jnp.pad(x[:, 0], (0, n_pad - n)).reshape(rn, 128)

    deg_k = _make_deg_kernel(n_blocks, n_pad, nc, ns)
    agg_k = _make_agg_kernel(n_blocks, n_pad, nc, ns)

    degp = deg_k(e3).reshape(nc, rn, 128)

    smem = pl.BlockSpec(memory_space=pltpu.SMEM)
    f32 = jnp.float32
    dinv2d, u2d = pl.pallas_call(
        _stage1_body,
        out_shape=[jax.ShapeDtypeStruct((rn, 128), f32)] * 2,
    )(degp, x2d)

    s1p = agg_k(e3, u2d.reshape(n_pad)).reshape(nc, rn, 128)

    t2d, v2d = pl.pallas_call(
        _stage2_body,
        in_specs=[pl.BlockSpec(), pl.BlockSpec(), pl.BlockSpec(),
                  smem, smem, smem],
        out_shape=[jax.ShapeDtypeStruct((rn, 128), f32)] * 2,
    )(s1p, x2d, dinv2d, W1.reshape(1, 16), b1.reshape(1, 16), W2.reshape(1, 16))

    s2p = agg_k(e3, v2d.reshape(n_pad)).reshape(nc, rn, 128)

    o2d = pl.pallas_call(
        _stage3_body,
        in_specs=[pl.BlockSpec(), pl.BlockSpec(), pl.BlockSpec(), smem],
        out_shape=jax.ShapeDtypeStruct((rn, 128), f32),
    )(s2p, t2d, dinv2d, b2)

    return o2d.reshape(n_pad)[:n]


# speedup vs baseline: 533.0673x; 1.1468x over previous
import functools

import jax
import jax.numpy as jnp
from jax import lax
from jax.experimental import pallas as pl
from jax.experimental.pallas import tpu as pltpu
from jax.experimental.pallas import tpu_sc as plsc

ROW = 128
BLK = 8
DEPTH = 8
LEAD = 4
NSLOT = 2 * BLK
CHUNK = BLK * ROW


def _tile_blocks(n_blocks, nw):
    base = n_blocks // nw
    extra = n_blocks - base * nw
    return base, extra


def _block_loop(base, extra, wid, issue_dma, wait_dma, process, wait_scat):
    def do_block(k, p, parity, skip_slot_wait=False, guard_dma=False):
        def issue():
            issue_dma(k + LEAD, (p + LEAD) % DEPTH)

        if guard_dma:
            nblk_w = base + jnp.where(wid < extra, 1, 0)
            pl.when(k + LEAD < nblk_w)(issue)
        else:
            issue()
        wait_dma(p)
        process(p, parity, skip_slot_wait)

    for k in range(LEAD):
        issue_dma(jnp.int32(k), k)
    yield

    for k in range(LEAD):
        do_block(jnp.int32(k), k, k % 2, skip_slot_wait=k < 2)

    def body(t, carry):
        for pp in range(DEPTH):
            do_block(t * DEPTH + LEAD + pp, (LEAD + pp) % DEPTH, pp % 2)
        return carry

    n_main = (base - LEAD) // DEPTH
    lax.fori_loop(0, n_main, body, 0)

    for q in range(LEAD + n_main * DEPTH, base + 1):
        blk_fn = functools.partial(do_block, jnp.int32(q), q % DEPTH, q % 2,
                                   False, True)
        if q == base:
            pl.when(wid < extra)(blk_fn)
        else:
            blk_fn()

    for sv in range(NSLOT):
        wait_scat(sv)
    yield


def _make_deg_kernel(n_blocks, n_pad, nc, ns):
    nw = nc * ns
    base, extra = _tile_blocks(n_blocks, nw)
    stripe = n_pad // ns
    mesh = plsc.VectorSubcoreMesh(core_axis_name="c", subcore_axis_name="s")

    @functools.partial(
        pl.kernel,
        mesh=mesh,
        out_type=jax.ShapeDtypeStruct((nc * n_pad,), jnp.float32),
        scratch_types=(
            [pltpu.VMEM((CHUNK,), jnp.int32) for _ in range(DEPTH)]
            + [pltpu.VMEM((ROW,), jnp.float32),
               pltpu.VMEM((stripe,), jnp.float32),
               pltpu.VMEM_SHARED((n_pad,), jnp.float32)]
            + [pltpu.SemaphoreType.DMA for _ in range(DEPTH + NSLOT)]
        ),
    )
    def deg_kernel(e_hbm, out_hbm, *scratch):
        dbuf = scratch[:DEPTH]
        ones, zbuf, acc = scratch[DEPTH:DEPTH + 3]
        edma = scratch[DEPTH + 3:DEPTH + 3 + DEPTH]
        ssem = scratch[DEPTH + 3 + DEPTH:]

        c = lax.axis_index("c")
        s = lax.axis_index("s")
        wid = s * nc + c

        for g in range(ROW // 16):
            ones[pl.ds(g * 16, 16)] = jnp.full((16,), 1.0, jnp.float32)

        def fill_zero(i, carry):
            zbuf[pl.ds(i * 16, 16)] = jnp.zeros((16,), jnp.float32)
            return carry

        lax.fori_loop(0, stripe // 16, fill_zero, 0)
        pltpu.sync_copy(zbuf, acc.at[pl.ds(s * stripe, stripe)])

        def issue_dma(k, p):
            blk = wid + k * nw
            pltpu.async_copy(e_hbm.at[1, pl.ds(blk * CHUNK, CHUNK)], dbuf[p],
                             edma[p])

        def wait_dma(p):
            pltpu.make_async_copy(e_hbm.at[1, pl.ds(0, CHUNK)], dbuf[p],
                                  edma[p]).wait()

        def wait_scat(sv):
            pltpu.make_async_copy(ones, acc.at[dbuf[0].at[pl.ds(0, ROW)]],
                                  ssem[sv]).wait()

        def process(p, parity, skip_slot_wait):
            for r in range(BLK):
                sv = parity * BLK + r
                if not skip_slot_wait:
                    wait_scat(sv)
                pltpu.async_copy(ones, acc.at[dbuf[p].at[pl.ds(r * ROW, ROW)]],
                                 ssem[sv], add=True)

        loop = _block_loop(base, extra, wid, issue_dma, wait_dma, process,
                           wait_scat)
        next(loop)
        plsc.subcore_barrier()
        next(loop, None)

        plsc.subcore_barrier()
        pltpu.sync_copy(acc.at[pl.ds(s * stripe, stripe)], zbuf)
        pltpu.sync_copy(zbuf, out_hbm.at[pl.ds(c * n_pad + s * stripe, stripe)])

    return deg_kernel


def _make_agg_kernel(n_blocks, n_pad, nc, ns):
    nw = nc * ns
    base, extra = _tile_blocks(n_blocks, nw)
    stripe = n_pad // ns
    mesh = plsc.VectorSubcoreMesh(core_axis_name="c", subcore_axis_name="s")

    @functools.partial(
        pl.kernel,
        mesh=mesh,
        out_type=jax.ShapeDtypeStruct((nc * n_pad,), jnp.float32),
        compiler_params=pltpu.CompilerParams(needs_layout_passes=False),
        scratch_types=(
            [pltpu.VMEM((n_pad,), jnp.float32)]
            + [pltpu.VMEM((2, CHUNK), jnp.int32) for _ in range(DEPTH)]
            + [pltpu.VMEM((ROW,), jnp.float32) for _ in range(NSLOT)]
            + [pltpu.VMEM((stripe,), jnp.float32),
               pltpu.VMEM_SHARED((n_pad,), jnp.float32)]
            + [pltpu.SemaphoreType.DMA for _ in range(1 + DEPTH + NSLOT)]
        ),
    )
    def agg_kernel(e_hbm, u_hbm, out_hbm, *scratch):
        utab = scratch[0]
        ebuf = scratch[1:1 + DEPTH]
        vbuf = scratch[1 + DEPTH:1 + DEPTH + NSLOT]
        zbuf, acc = scratch[1 + DEPTH + NSLOT:3 + DEPTH + NSLOT]
        usem = scratch[3 + DEPTH + NSLOT]
        edma = scratch[4 + DEPTH + NSLOT:4 + 2 * DEPTH + NSLOT]
        ssem = scratch[4 + 2 * DEPTH + NSLOT:]

        c = lax.axis_index("c")
        s = lax.axis_index("s")
        wid = s * nc + c

        pltpu.async_copy(u_hbm, utab, usem)

        def fill_zero(i, carry):
            zbuf[pl.ds(i * 16, 16)] = jnp.zeros((16,), jnp.float32)
            return carry

        lax.fori_loop(0, stripe // 16, fill_zero, 0)
        pltpu.sync_copy(zbuf, acc.at[pl.ds(s * stripe, stripe)])

        def issue_dma(k, p):
            blk = wid + k * nw
            pltpu.async_copy(e_hbm.at[:, pl.ds(blk * CHUNK, CHUNK)], ebuf[p],
                             edma[p])

        def wait_dma(p):
            pltpu.make_async_copy(e_hbm.at[:, pl.ds(0, CHUNK)], ebuf[p],
                                  edma[p]).wait()

        def wait_scat(sv):
            pltpu.make_async_copy(vbuf[sv],
                                  acc.at[ebuf[0].at[1, pl.ds(0, ROW)]],
                                  ssem[sv]).wait()

        def process(p, parity, skip_slot_wait):
            for r in range(BLK):
                sv = parity * BLK + r
                if not skip_slot_wait:
                    wait_scat(sv)
                for g in range(ROW // 16):
                    idx = ebuf[p][0, pl.ds(r * ROW + g * 16, 16)]
                    vbuf[sv][pl.ds(g * 16, 16)] = plsc.load_gather(
                        utab, [idx])
                pltpu.async_copy(vbuf[sv],
                                 acc.at[ebuf[p].at[1, pl.ds(r * ROW, ROW)]],
                                 ssem[sv], add=True)

        loop = _block_loop(base, extra, wid, issue_dma, wait_dma, process,
                           wait_scat)
        next(loop)
        pltpu.make_async_copy(u_hbm, utab, usem).wait()
        plsc.subcore_barrier()
        next(loop, None)

        plsc.subcore_barrier()
        pltpu.sync_copy(acc.at[pl.ds(s * stripe, stripe)], zbuf)
        pltpu.sync_copy(zbuf, out_hbm.at[pl.ds(c * n_pad + s * stripe, stripe)])

    return agg_kernel


def _stage1_body(p_ref, x_ref, dinv_ref, u_ref):
    deg = p_ref[0] + p_ref[1] + 1.0
    dinv = lax.rsqrt(deg)
    dinv_ref[...] = dinv
    u_ref[...] = x_ref[...] * dinv


def _stage2_body(p_ref, x_ref, dinv_ref, w1_ref, b1_ref, w2_ref, t_ref, v_ref):
    dinv = dinv_ref[...]
    s1 = dinv * (p_ref[0] + p_ref[1]) + x_ref[...] * dinv * dinv
    t = jnp.zeros_like(s1)
    for k in range(16):
        t = t + jnp.maximum(s1 * w1_ref[0, k] + b1_ref[0, k], 0.0) * w2_ref[0, k]
    t_ref[...] = t
    v_ref[...] = t * dinv


def _stage3_body(p_ref, t_ref, dinv_ref, b2_ref, o_ref):
    dinv = dinv_ref[...]
    t = t_ref[...]
    o_ref[...] = dinv * (p_ref[0] + p_ref[1]) + t * dinv * dinv + b2_ref[0]


def kernel(x, edge_index, W1, b1, W2, b2):
    n = x.shape[0]
    e = edge_index.shape[1]
    assert e % CHUNK == 0
    n_blocks = e // CHUNK

    info = plsc.get_sparse_core_info()
    nc, ns = info.num_cores, info.num_subcores

    stripe = -(-n // (ns * 8)) * 8
    n_pad = ns * stripe
    rn = n_pad // 128

    e3 = edge_index.astype(jnp.int32)
    x2d = jnp.pad(x[:, 0], (0, n_pad - n)).reshape(rn, 128)

    deg_k = _make_deg_kernel(n_blocks, n_pad, nc, ns)
    agg_k = _make_agg_kernel(n_blocks, n_pad, nc, ns)

    degp = deg_k(e3).reshape(nc, rn, 128)

    smem = pl.BlockSpec(memory_space=pltpu.SMEM)
    f32 = jnp.float32
    dinv2d, u2d = pl.pallas_call(
        _stage1_body,
        out_shape=[jax.ShapeDtypeStruct((rn, 128), f32)] * 2,
    )(degp, x2d)

    s1p = agg_k(e3, u2d.reshape(n_pad)).reshape(nc, rn, 128)

    t2d, v2d = pl.pallas_call(
        _stage2_body,
        in_specs=[pl.BlockSpec(), pl.BlockSpec(), pl.BlockSpec(),
                  smem, smem, smem],
        out_shape=[jax.ShapeDtypeStruct((rn, 128), f32)] * 2,
    )(s1p, x2d, dinv2d, W1.reshape(1, 16), b1.reshape(1, 16), W2.reshape(1, 16))

    s2p = agg_k(e3, v2d.reshape(n_pad)).reshape(nc, rn, 128)

    o2d = pl.pallas_call(
        _stage3_body,
        in_specs=[pl.BlockSpec(), pl.BlockSpec(), pl.BlockSpec(), smem],
        out_shape=jax.ShapeDtypeStruct((rn, 128), f32),
    )(s2p, t2d, dinv2d, b2)

    return o2d.reshape(n_pad)[:n]


# speedup vs baseline: 548.6709x; 1.0293x over previous
import functools

import jax
import jax.numpy as jnp
from jax import lax
from jax.experimental import pallas as pl
from jax.experimental.pallas import tpu as pltpu
from jax.experimental.pallas import tpu_sc as plsc

ROW = 128
BLK = 8
DEPTH = 8
LEAD = 4
NSLOT = 2
CHUNK = BLK * ROW


def _tile_blocks(n_blocks, nw):
    base = n_blocks // nw
    extra = n_blocks - base * nw
    return base, extra


def _block_loop(base, extra, wid, issue_dma, wait_dma, process, wait_scat):
    def do_block(k, p, slot, skip_slot_wait=False, guard_dma=False):
        def issue():
            issue_dma(k + LEAD, (p + LEAD) % DEPTH)

        if guard_dma:
            nblk_w = base + jnp.where(wid < extra, 1, 0)
            pl.when(k + LEAD < nblk_w)(issue)
        else:
            issue()
        wait_dma(p)
        process(p, slot, skip_slot_wait)

    for k in range(LEAD):
        issue_dma(jnp.int32(k), k)
    yield

    for k in range(LEAD):
        do_block(jnp.int32(k), k, k % NSLOT, skip_slot_wait=k < NSLOT)

    def body(t, carry):
        for pp in range(DEPTH):
            do_block(t * DEPTH + LEAD + pp, (LEAD + pp) % DEPTH,
                     (LEAD + pp) % NSLOT)
        return carry

    n_main = (base - LEAD) // DEPTH
    lax.fori_loop(0, n_main, body, 0)

    for q in range(LEAD + n_main * DEPTH, base + 1):
        blk_fn = functools.partial(do_block, jnp.int32(q), q % DEPTH,
                                   q % NSLOT, False, True)
        if q == base:
            pl.when(wid < extra)(blk_fn)
        else:
            blk_fn()

    for sv in range(NSLOT):
        wait_scat(sv)
    yield


def _make_deg_kernel(n_blocks, n_pad, nc, ns):
    nw = nc * ns
    base, extra = _tile_blocks(n_blocks, nw)
    stripe = n_pad // ns
    mesh = plsc.VectorSubcoreMesh(core_axis_name="c", subcore_axis_name="s")

    @functools.partial(
        pl.kernel,
        mesh=mesh,
        out_type=jax.ShapeDtypeStruct((nc * n_pad,), jnp.float32),
        scratch_types=(
            [pltpu.VMEM((CHUNK,), jnp.int32) for _ in range(DEPTH)]
            + [pltpu.VMEM((CHUNK,), jnp.float32),
               pltpu.VMEM((stripe,), jnp.float32),
               pltpu.VMEM_SHARED((n_pad,), jnp.float32)]
            + [pltpu.SemaphoreType.DMA for _ in range(DEPTH + NSLOT)]
        ),
    )
    def deg_kernel(e_hbm, out_hbm, *scratch):
        dbuf = scratch[:DEPTH]
        ones, zbuf, acc = scratch[DEPTH:DEPTH + 3]
        edma = scratch[DEPTH + 3:DEPTH + 3 + DEPTH]
        ssem = scratch[DEPTH + 3 + DEPTH:]

        c = lax.axis_index("c")
        s = lax.axis_index("s")
        wid = s * nc + c

        def fill_one(i, carry):
            ones[pl.ds(i * 16, 16)] = jnp.full((16,), 1.0, jnp.float32)
            return carry

        lax.fori_loop(0, CHUNK // 16, fill_one, 0)

        def fill_zero(i, carry):
            zbuf[pl.ds(i * 16, 16)] = jnp.zeros((16,), jnp.float32)
            return carry

        lax.fori_loop(0, stripe // 16, fill_zero, 0)
        pltpu.sync_copy(zbuf, acc.at[pl.ds(s * stripe, stripe)])

        def issue_dma(k, p):
            blk = wid + k * nw
            pltpu.async_copy(e_hbm.at[1, pl.ds(blk * CHUNK, CHUNK)], dbuf[p],
                             edma[p])

        def wait_dma(p):
            pltpu.make_async_copy(e_hbm.at[1, pl.ds(0, CHUNK)], dbuf[p],
                                  edma[p]).wait()

        def wait_scat(sv):
            pltpu.make_async_copy(ones, acc.at[dbuf[0].at[pl.ds(0, CHUNK)]],
                                  ssem[sv]).wait()

        def process(p, sv, skip_slot_wait):
            if not skip_slot_wait:
                wait_scat(sv)
            pltpu.async_copy(ones, acc.at[dbuf[p].at[pl.ds(0, CHUNK)]],
                             ssem[sv], add=True)

        loop = _block_loop(base, extra, wid, issue_dma, wait_dma, process,
                           wait_scat)
        next(loop)
        plsc.subcore_barrier()
        next(loop, None)

        plsc.subcore_barrier()
        pltpu.sync_copy(acc.at[pl.ds(s * stripe, stripe)], zbuf)
        pltpu.sync_copy(zbuf, out_hbm.at[pl.ds(c * n_pad + s * stripe, stripe)])

    return deg_kernel


def _make_agg_kernel(n_blocks, n_pad, nc, ns):
    nw = nc * ns
    base, extra = _tile_blocks(n_blocks, nw)
    stripe = n_pad // ns
    mesh = plsc.VectorSubcoreMesh(core_axis_name="c", subcore_axis_name="s")

    @functools.partial(
        pl.kernel,
        mesh=mesh,
        out_type=jax.ShapeDtypeStruct((nc * n_pad,), jnp.float32),
        compiler_params=pltpu.CompilerParams(needs_layout_passes=False),
        scratch_types=(
            [pltpu.VMEM((n_pad,), jnp.float32)]
            + [pltpu.VMEM((CHUNK,), jnp.int32) for _ in range(2 * DEPTH)]
            + [pltpu.VMEM((CHUNK,), jnp.float32) for _ in range(NSLOT)]
            + [pltpu.VMEM((stripe,), jnp.float32),
               pltpu.VMEM_SHARED((n_pad,), jnp.float32)]
            + [pltpu.SemaphoreType.DMA for _ in range(1 + 2 * DEPTH + NSLOT)]
        ),
    )
    def agg_kernel(e_hbm, u_hbm, out_hbm, *scratch):
        utab = scratch[0]
        sbuf = scratch[1:1 + DEPTH]
        tbuf = scratch[1 + DEPTH:1 + 2 * DEPTH]
        base_v = 1 + 2 * DEPTH
        vbuf = scratch[base_v:base_v + NSLOT]
        zbuf, acc = scratch[base_v + NSLOT:base_v + NSLOT + 2]
        usem = scratch[base_v + NSLOT + 2]
        sdma = scratch[base_v + NSLOT + 3:base_v + NSLOT + 3 + DEPTH]
        tdma = scratch[base_v + NSLOT + 3 + DEPTH:base_v + NSLOT + 3 + 2 * DEPTH]
        ssem = scratch[base_v + NSLOT + 3 + 2 * DEPTH:]

        c = lax.axis_index("c")
        s = lax.axis_index("s")
        wid = s * nc + c

        pltpu.async_copy(u_hbm, utab, usem)

        def fill_zero(i, carry):
            zbuf[pl.ds(i * 16, 16)] = jnp.zeros((16,), jnp.float32)
            return carry

        lax.fori_loop(0, stripe // 16, fill_zero, 0)
        pltpu.sync_copy(zbuf, acc.at[pl.ds(s * stripe, stripe)])

        def issue_dma(k, p):
            blk = wid + k * nw
            pltpu.async_copy(e_hbm.at[0, pl.ds(blk * CHUNK, CHUNK)], sbuf[p],
                             sdma[p])
            pltpu.async_copy(e_hbm.at[1, pl.ds(blk * CHUNK, CHUNK)], tbuf[p],
                             tdma[p])

        def wait_dma(p):
            pltpu.make_async_copy(e_hbm.at[0, pl.ds(0, CHUNK)], sbuf[p],
                                  sdma[p]).wait()
            pltpu.make_async_copy(e_hbm.at[1, pl.ds(0, CHUNK)], tbuf[p],
                                  tdma[p]).wait()

        def wait_scat(sv):
            pltpu.make_async_copy(vbuf[sv],
                                  acc.at[tbuf[0].at[pl.ds(0, CHUNK)]],
                                  ssem[sv]).wait()

        def process(p, sv, skip_slot_wait):
            if not skip_slot_wait:
                wait_scat(sv)

            def gather_row(r, carry):
                for g in range(ROW // 16):
                    off = r * ROW + g * 16
                    idx = sbuf[p][pl.ds(off, 16)]
                    vbuf[sv][pl.ds(off, 16)] = plsc.load_gather(utab, [idx])
                return carry

            lax.fori_loop(0, BLK, gather_row, 0)
            pltpu.async_copy(vbuf[sv],
                             acc.at[tbuf[p].at[pl.ds(0, CHUNK)]],
                             ssem[sv], add=True)

        loop = _block_loop(base, extra, wid, issue_dma, wait_dma, process,
                           wait_scat)
        next(loop)
        pltpu.make_async_copy(u_hbm, utab, usem).wait()
        plsc.subcore_barrier()
        next(loop, None)

        plsc.subcore_barrier()
        pltpu.sync_copy(acc.at[pl.ds(s * stripe, stripe)], zbuf)
        pltpu.sync_copy(zbuf, out_hbm.at[pl.ds(c * n_pad + s * stripe, stripe)])

    return agg_kernel


def _stage1_body(p_ref, x_ref, dinv_ref, u_ref):
    deg = p_ref[0] + p_ref[1] + 1.0
    dinv = lax.rsqrt(deg)
    dinv_ref[...] = dinv
    u_ref[...] = x_ref[...] * dinv


def _stage2_body(p_ref, x_ref, dinv_ref, w1_ref, b1_ref, w2_ref, t_ref, v_ref):
    dinv = dinv_ref[...]
    s1 = dinv * (p_ref[0] + p_ref[1]) + x_ref[...] * dinv * dinv
    t = jnp.zeros_like(s1)
    for k in range(16):
        t = t + jnp.maximum(s1 * w1_ref[0, k] + b1_ref[0, k], 0.0) * w2_ref[0, k]
    t_ref[...] = t
    v_ref[...] = t * dinv


def _stage3_body(p_ref, t_ref, dinv_ref, b2_ref, o_ref):
    dinv = dinv_ref[...]
    t = t_ref[...]
    o_ref[...] = dinv * (p_ref[0] + p_ref[1]) + t * dinv * dinv + b2_ref[0]


def kernel(x, edge_index, W1, b1, W2, b2):
    n = x.shape[0]
    e = edge_index.shape[1]
    assert e % CHUNK == 0
    n_blocks = e // CHUNK

    info = plsc.get_sparse_core_info()
    nc, ns = info.num_cores, info.num_subcores

    stripe = -(-n // (ns * 8)) * 8
    n_pad = ns * stripe
    rn = n_pad // 128

    e3 = edge_index.astype(jnp.int32)
    x2d = jnp.pad(x[:, 0], (0, n_pad - n)).reshape(rn, 128)

    deg_k = _make_deg_kernel(n_blocks, n_pad, nc, ns)
    agg_k = _make_agg_kernel(n_blocks, n_pad, nc, ns)

    degp = deg_k(e3).reshape(nc, rn, 128)

    smem = pl.BlockSpec(memory_space=pltpu.SMEM)
    f32 = jnp.float32
    dinv2d, u2d = pl.pallas_call(
        _stage1_body,
        out_shape=[jax.ShapeDtypeStruct((rn, 128), f32)] * 2,
    )(degp, x2d)

    s1p = agg_k(e3, u2d.reshape(n_pad)).reshape(nc, rn, 128)

    t2d, v2d = pl.pallas_call(
        _stage2_body,
        in_specs=[pl.BlockSpec(), pl.BlockSpec(), pl.BlockSpec(),
                  smem, smem, smem],
        out_shape=[jax.ShapeDtypeStruct((rn, 128), f32)] * 2,
    )(s1p, x2d, dinv2d, W1.reshape(1, 16), b1.reshape(1, 16), W2.reshape(1, 16))

    s2p = agg_k(e3, v2d.reshape(n_pad)).reshape(nc, rn, 128)

    o2d = pl.pallas_call(
        _stage3_body,
        in_specs=[pl.BlockSpec(), pl.BlockSpec(), pl.BlockSpec(), smem],
        out_shape=jax.ShapeDtypeStruct((rn, 128), f32),
    )(s2p, t2d, dinv2d, b2)

    return o2d.reshape(n_pad)[:n]
